# baseline (speedup 1.0000x reference)
import jax
import jax.numpy as jnp
from jax.experimental import pallas as pl
from jax.experimental.pallas import tpu as pltpu

N_CODES = 8192
HALF = N_CODES // 2
DIM = 64
DEPTH = 4
ROWS = 8192
TILE_R = 256
NT = ROWS // TILE_R

_HI = jax.lax.Precision.HIGHEST
_DN = (((1,), (1,)), ((), ()))
_DN0 = (((1,), (0,)), ((), ()))


def _first_argmin(dist, col):
    m = jnp.min(dist, axis=1, keepdims=True)
    idx = jnp.min(jnp.where(dist == m, col, N_CODES), axis=1, keepdims=True)
    return m, idx


def _vq_body(x_ref, c0, c1, c2, c3, agg_ref, codes_ref, loss_ref):
    cbs = (c0, c1, c2, c3)
    x = x_ref[...]
    r = x
    agg = jnp.zeros_like(x)
    col = jax.lax.broadcasted_iota(jnp.int32, (TILE_R, HALF), 1)
    ones_row = jnp.ones((1, DIM), jnp.float32)
    idx_cols = []
    loss_vals = []
    for d in range(DEPTH):
        cb = cbs[d][...]
        cbb = cb.astype(jnp.bfloat16)
        rb = r.astype(jnp.bfloat16)
        s = jax.lax.dot_general(rb, cbb, _DN,
                                preferred_element_type=jnp.float32)
        cbn = jax.lax.dot_general(ones_row, cb * cb, _DN,
                                  precision=_HI)
        in_n = jnp.sum(r * r, axis=1, keepdims=True)
        dist = (in_n + cbn) - 2.0 * s
        m0, i0 = _first_argmin(dist[:, :HALF], col)
        m1, i1 = _first_argmin(dist[:, HALF:], col)
        i1 = i1 + HALF
        if d in (0, 3):
            take1 = m1 < m0.astype(jnp.bfloat16).astype(jnp.float32)
        else:
            take1 = m1 < m0
        idx = jnp.where(take1, i1, i0)
        onehot = jnp.concatenate(
            [(col == idx).astype(jnp.bfloat16),
             (col == (idx - HALF)).astype(jnp.bfloat16)], axis=1)
        cb_hi = cbb
        hi_f = cb_hi.astype(jnp.float32)
        cb_lo = (cb - hi_f).astype(jnp.bfloat16)
        cb_ll = (cb - hi_f - cb_lo.astype(jnp.float32)).astype(jnp.bfloat16)
        q = jax.lax.dot_general(onehot, cb_hi, _DN0,
                                preferred_element_type=jnp.float32)
        q = q + jax.lax.dot_general(onehot, cb_lo, _DN0,
                                    preferred_element_type=jnp.float32)
        q = q + jax.lax.dot_general(onehot, cb_ll, _DN0,
                                    preferred_element_type=jnp.float32)
        r = r - q
        agg = agg + q
        idx_cols.append(idx)
        loss_vals.append(
            jnp.sum(jnp.sum(r * r, axis=1, keepdims=True), axis=0,
                    keepdims=True))
    codes_ref[...] = jnp.concatenate(idx_cols, axis=1)
    loss_ref[0] = jnp.concatenate(loss_vals, axis=1)
    agg_ref[...] = x + (agg - x)


def kernel(x, C0, C1, C2, C3):
    xf = x.reshape(ROWS, DIM)
    cbs = [C[:-1] for C in (C0, C1, C2, C3)]

    full = lambda shape: pl.BlockSpec(shape, lambda i: (0,) * len(shape))
    agg, codes, loss_sums = pl.pallas_call(
        _vq_body,
        grid=(NT,),
        in_specs=[pl.BlockSpec((TILE_R, DIM), lambda i: (i, 0))]
        + [full((N_CODES, DIM))] * DEPTH,
        out_specs=[
            pl.BlockSpec((TILE_R, DIM), lambda i: (i, 0)),
            pl.BlockSpec((TILE_R, DEPTH), lambda i: (i, 0)),
            pl.BlockSpec((1, 1, DEPTH), lambda i: (i, 0, 0)),
        ],
        out_shape=[
            jax.ShapeDtypeStruct((ROWS, DIM), jnp.float32),
            jax.ShapeDtypeStruct((ROWS, DEPTH), jnp.int32),
            jax.ShapeDtypeStruct((NT, 1, DEPTH), jnp.float32),
        ],
        compiler_params=pltpu.CompilerParams(
            dimension_semantics=("parallel",)),
    )(xf, *cbs)

    quants = agg.reshape(x.shape)
    codes = codes.reshape(x.shape[:-1] + (DEPTH,))
    loss = jnp.mean(jnp.sum(loss_sums, axis=(0, 1)) / (ROWS * DIM))
    return quants, loss, codes


# speedup vs baseline: 3.0774x; 3.0774x over previous
import functools

import jax
import jax.numpy as jnp
from jax import lax
from jax.experimental import pallas as pl
from jax.experimental.pallas import tpu as pltpu
from jax.experimental.pallas import tpu_sc as plsc

N_CODES = 8192
HALF = N_CODES // 2
DIM = 64
DEPTH = 4
ROWS = 8192
TILE_R = 256
NT = ROWS // TILE_R
NW = 32
B_PER_W = ROWS // NW

_HI = jax.lax.Precision.HIGHEST
_DN = (((1,), (1,)), ((), ()))


def _first_argmin(dist, col):
    m = jnp.min(dist, axis=1, keepdims=True)
    idx = jnp.min(jnp.where(dist == m, col, N_CODES), axis=1, keepdims=True)
    return m, idx


def _choose(r, cbb_ref, cbn_ref, windowed):
    s = jax.lax.dot_general(r.astype(jnp.bfloat16), cbb_ref[...], _DN,
                            preferred_element_type=jnp.float32)
    in_n = jnp.sum(r * r, axis=1, keepdims=True)
    dist = (in_n + cbn_ref[...]) - 2.0 * s
    col = jax.lax.broadcasted_iota(jnp.int32, (TILE_R, HALF), 1)
    m0, i0 = _first_argmin(dist[:, :HALF], col)
    m1, i1 = _first_argmin(dist[:, HALF:], col)
    i1 = i1 + HALF
    if windowed:
        take1 = m1 < m0.astype(jnp.bfloat16).astype(jnp.float32)
    else:
        take1 = m1 < m0
    return jnp.where(take1, i1, i0)


def _depth0_body(x_ref, cbb_ref, cbn_ref, idx_ref):
    idx_ref[...] = _choose(x_ref[...], cbb_ref, cbn_ref, True)


def _depth_body(windowed, rprev_ref, qprev_ref, cbb_ref, cbn_ref,
                idx_ref, r_ref, loss_ref):
    r = rprev_ref[...] - qprev_ref[:, :DIM]
    r_ref[...] = r
    loss_ref[0] = jnp.sum(jnp.sum(r * r, axis=1, keepdims=True),
                          axis=0, keepdims=True)
    idx_ref[...] = _choose(r, cbb_ref, cbn_ref, windowed)


def _final_body(x_ref, rprev_ref, qprev_ref, out_ref, loss_ref):
    r = rprev_ref[...] - qprev_ref[:, :DIM]
    loss_ref[0] = jnp.sum(jnp.sum(r * r, axis=1, keepdims=True),
                          axis=0, keepdims=True)
    x = x_ref[...]
    agg = x - r
    out_ref[...] = x + (agg - x)


def _norms_body(c0, c1, c2, c3, out_ref):
    ones_row = jnp.ones((1, DIM), jnp.float32)
    for d, cr in enumerate((c0, c1, c2, c3)):
        cb = cr[...]
        out_ref[d:d + 1, :] = jax.lax.dot_general(
            ones_row, cb * cb, _DN, precision=_HI)


_row_spec = pl.BlockSpec((TILE_R, DIM), lambda i: (i, 0))
_q_spec = pl.BlockSpec((TILE_R, 2 * DIM), lambda i: (i, 0))
_idx_spec = pl.BlockSpec((TILE_R, 1), lambda i: (i, 0))
_loss_spec = pl.BlockSpec((1, 1, 1), lambda i: (i, 0, 0))
_cbb_spec = pl.BlockSpec((N_CODES, DIM), lambda i: (0, 0))
_cbn_spec = pl.BlockSpec((1, N_CODES), lambda i: (0, 0))
_PAR = pltpu.CompilerParams(dimension_semantics=("parallel",))

_idx_shape = jax.ShapeDtypeStruct((ROWS, 1), jnp.int32)
_row_shape = jax.ShapeDtypeStruct((ROWS, DIM), jnp.float32)
_loss_shape = jax.ShapeDtypeStruct((NT, 1, 1), jnp.float32)


def _sc_gather(table, idx):
    mesh = plsc.VectorSubcoreMesh(core_axis_name="c", subcore_axis_name="s")

    @functools.partial(
        pl.kernel, mesh=mesh,
        out_type=jax.ShapeDtypeStruct((ROWS, 2 * DIM), jnp.float32),
        scratch_types=[
            pltpu.VMEM((B_PER_W,), jnp.int32),
            pltpu.VMEM((B_PER_W, 2 * DIM), jnp.float32),
            pltpu.SemaphoreType.DMA,
        ],
    )
    def k(table_hbm, idx_hbm, out_hbm, idx_v, rows_v, sem):
        wid = lax.axis_index("s") * 2 + lax.axis_index("c")
        base = wid * B_PER_W
        pltpu.sync_copy(idx_hbm.at[pl.ds(base, B_PER_W)], idx_v)
        pltpu.async_copy(table_hbm.at[idx_v], rows_v, sem).wait()
        pltpu.sync_copy(rows_v, out_hbm.at[pl.ds(base, B_PER_W)])

    return k(table, idx)


def kernel(x, C0, C1, C2, C3):
    xf = x.reshape(ROWS, DIM)
    tables = [jnp.pad(C, ((0, 0), (0, DIM))) for C in (C0, C1, C2, C3)]
    cbs = [C[:-1] for C in (C0, C1, C2, C3)]
    cbbs = [c.astype(jnp.bfloat16) for c in cbs]

    cbn_all = pl.pallas_call(
        _norms_body,
        in_specs=[pl.BlockSpec((N_CODES, DIM), lambda: (0, 0))] * DEPTH,
        out_specs=pl.BlockSpec((DEPTH, N_CODES), lambda: (0, 0)),
        out_shape=jax.ShapeDtypeStruct((DEPTH, N_CODES), jnp.float32),
    )(*cbs)
    cbns = [jax.lax.slice(cbn_all, (d, 0), (d + 1, N_CODES))
            for d in range(DEPTH)]

    idx0 = pl.pallas_call(
        _depth0_body,
        grid=(NT,),
        in_specs=[_row_spec, _cbb_spec, _cbn_spec],
        out_specs=_idx_spec,
        out_shape=_idx_shape,
        compiler_params=_PAR,
    )(xf, cbbs[0], cbns[0])
    q = _sc_gather(tables[0], idx0.reshape(ROWS))

    rprev = xf
    idxs = [idx0]
    loss_sums = []
    for d in (1, 2, 3):
        idx_d, rprev, ls = pl.pallas_call(
            functools.partial(_depth_body, d in (0, 3)),
            grid=(NT,),
            in_specs=[_row_spec, _q_spec, _cbb_spec, _cbn_spec],
            out_specs=[_idx_spec, _row_spec, _loss_spec],
            out_shape=[_idx_shape, _row_shape, _loss_shape],
            compiler_params=_PAR,
        )(rprev, q, cbbs[d], cbns[d])
        q = _sc_gather(tables[d], idx_d.reshape(ROWS))
        idxs.append(idx_d)
        loss_sums.append(ls)

    quants, ls3 = pl.pallas_call(
        _final_body,
        grid=(NT,),
        in_specs=[_row_spec, _row_spec, _q_spec],
        out_specs=[_row_spec, _loss_spec],
        out_shape=[_row_shape, _loss_shape],
        compiler_params=_PAR,
    )(xf, rprev, q)
    loss_sums.append(ls3)

    quants = quants.reshape(x.shape)
    codes = jnp.concatenate(idxs, axis=1).reshape(x.shape[:-1] + (DEPTH,))
    sums = jnp.stack([jnp.sum(l) for l in loss_sums])
    loss = jnp.mean(sums / (ROWS * DIM))
    return quants, loss, codes


# speedup vs baseline: 3.0816x; 1.0014x over previous
import functools

import jax
import jax.numpy as jnp
from jax import lax
from jax.experimental import pallas as pl
from jax.experimental.pallas import tpu as pltpu
from jax.experimental.pallas import tpu_sc as plsc

N_CODES = 8192
HALF = N_CODES // 2
DIM = 64
DEPTH = 4
ROWS = 8192
TILE_R = 256
NT = ROWS // TILE_R
NW = 32
B_PER_W = ROWS // NW

_HI = jax.lax.Precision.HIGHEST
_DN = (((1,), (1,)), ((), ()))


def _first_argmin(dist, col):
    m = jnp.min(dist, axis=1, keepdims=True)
    idx = jnp.min(jnp.where(dist == m, col, N_CODES), axis=1, keepdims=True)
    return m, idx


def _choose(r, cbb_ref, cbn_ref, windowed):
    s = jax.lax.dot_general(r.astype(jnp.bfloat16), cbb_ref[...], _DN,
                            preferred_element_type=jnp.float32)
    in_n = jnp.sum(r * r, axis=1, keepdims=True)
    dist = (in_n + cbn_ref[...]) - 2.0 * s
    col = jax.lax.broadcasted_iota(jnp.int32, (TILE_R, HALF), 1)
    m0, i0 = _first_argmin(dist[:, :HALF], col)
    m1, i1 = _first_argmin(dist[:, HALF:], col)
    i1 = i1 + HALF
    if windowed:
        take1 = m1 < m0.astype(jnp.bfloat16).astype(jnp.float32)
    else:
        take1 = m1 < m0
    return jnp.where(take1, i1, i0)


def _depth0_body(x_ref, cbb_ref, cbn_ref, idx_ref):
    idx_ref[...] = _choose(x_ref[...], cbb_ref, cbn_ref, True)


def _depth_body(windowed, rprev_ref, qprev_ref, cbb_ref, cbn_ref,
                idx_ref, r_ref, loss_ref):
    r = rprev_ref[...] - qprev_ref[:, :DIM]
    r_ref[...] = r
    loss_ref[0] = jnp.sum(jnp.sum(r * r, axis=1, keepdims=True),
                          axis=0, keepdims=True)
    idx_ref[...] = _choose(r, cbb_ref, cbn_ref, windowed)


def _final_body(x_ref, rprev_ref, qprev_ref, out_ref, loss_ref):
    r = rprev_ref[...] - qprev_ref[:, :DIM]
    loss_ref[0] = jnp.sum(jnp.sum(r * r, axis=1, keepdims=True),
                          axis=0, keepdims=True)
    x = x_ref[...]
    agg = x - r
    out_ref[...] = x + (agg - x)


def _norms_body(c0, c1, c2, c3, out_ref):
    ones_row = jnp.ones((1, DIM), jnp.float32)
    for d, cr in enumerate((c0, c1, c2, c3)):
        cb = cr[...]
        out_ref[d:d + 1, :] = jax.lax.dot_general(
            ones_row, cb * cb, _DN, precision=_HI)


_row_spec = pl.BlockSpec((TILE_R, DIM), lambda i: (i, 0))
_q_spec = pl.BlockSpec((TILE_R, 2 * DIM), lambda i: (i, 0))
_idx_spec = pl.BlockSpec((TILE_R, 1), lambda i: (i, 0))
_loss_spec = pl.BlockSpec((1, 1, 1), lambda i: (i, 0, 0))
_cbb_spec = pl.BlockSpec((N_CODES, DIM), lambda i: (0, 0))
_cbn_spec = pl.BlockSpec((1, N_CODES), lambda i: (0, 0))
_PAR = pltpu.CompilerParams(dimension_semantics=("arbitrary",))

_idx_shape = jax.ShapeDtypeStruct((ROWS, 1), jnp.int32)
_row_shape = jax.ShapeDtypeStruct((ROWS, DIM), jnp.float32)
_loss_shape = jax.ShapeDtypeStruct((NT, 1, 1), jnp.float32)


def _sc_gather(table, idx):
    mesh = plsc.VectorSubcoreMesh(core_axis_name="c", subcore_axis_name="s")

    @functools.partial(
        pl.kernel, mesh=mesh,
        out_type=jax.ShapeDtypeStruct((ROWS, 2 * DIM), jnp.float32),
        scratch_types=[
            pltpu.VMEM((B_PER_W,), jnp.int32),
            pltpu.VMEM((B_PER_W, 2 * DIM), jnp.float32),
            pltpu.SemaphoreType.DMA,
        ],
    )
    def k(table_hbm, idx_hbm, out_hbm, idx_v, rows_v, sem):
        wid = lax.axis_index("s") * 2 + lax.axis_index("c")
        base = wid * B_PER_W
        pltpu.sync_copy(idx_hbm.at[pl.ds(base, B_PER_W)], idx_v)
        pltpu.async_copy(table_hbm.at[idx_v], rows_v, sem).wait()
        pltpu.sync_copy(rows_v, out_hbm.at[pl.ds(base, B_PER_W)])

    return k(table, idx)


def kernel(x, C0, C1, C2, C3):
    xf = x.reshape(ROWS, DIM)
    tables = [jnp.pad(C, ((0, 0), (0, DIM))) for C in (C0, C1, C2, C3)]
    cbs = [C[:-1] for C in (C0, C1, C2, C3)]
    cbbs = [c.astype(jnp.bfloat16) for c in cbs]

    cbn_all = pl.pallas_call(
        _norms_body,
        in_specs=[pl.BlockSpec((N_CODES, DIM), lambda: (0, 0))] * DEPTH,
        out_specs=pl.BlockSpec((DEPTH, N_CODES), lambda: (0, 0)),
        out_shape=jax.ShapeDtypeStruct((DEPTH, N_CODES), jnp.float32),
    )(*cbs)
    cbns = [jax.lax.slice(cbn_all, (d, 0), (d + 1, N_CODES))
            for d in range(DEPTH)]

    idx0 = pl.pallas_call(
        _depth0_body,
        grid=(NT,),
        in_specs=[_row_spec, _cbb_spec, _cbn_spec],
        out_specs=_idx_spec,
        out_shape=_idx_shape,
        compiler_params=_PAR,
    )(xf, cbbs[0], cbns[0])
    q = _sc_gather(tables[0], idx0.reshape(ROWS))

    rprev = xf
    idxs = [idx0]
    loss_sums = []
    for d in (1, 2, 3):
        idx_d, rprev, ls = pl.pallas_call(
            functools.partial(_depth_body, d in (0, 3)),
            grid=(NT,),
            in_specs=[_row_spec, _q_spec, _cbb_spec, _cbn_spec],
            out_specs=[_idx_spec, _row_spec, _loss_spec],
            out_shape=[_idx_shape, _row_shape, _loss_shape],
            compiler_params=_PAR,
        )(rprev, q, cbbs[d], cbns[d])
        q = _sc_gather(tables[d], idx_d.reshape(ROWS))
        idxs.append(idx_d)
        loss_sums.append(ls)

    quants, ls3 = pl.pallas_call(
        _final_body,
        grid=(NT,),
        in_specs=[_row_spec, _row_spec, _q_spec],
        out_specs=[_row_spec, _loss_spec],
        out_shape=[_row_shape, _loss_shape],
        compiler_params=_PAR,
    )(xf, rprev, q)
    loss_sums.append(ls3)

    quants = quants.reshape(x.shape)
    codes = jnp.concatenate(idxs, axis=1).reshape(x.shape[:-1] + (DEPTH,))
    sums = jnp.stack([jnp.sum(l) for l in loss_sums])
    loss = jnp.mean(sums / (ROWS * DIM))
    return quants, loss, codes
